# baseline (speedup 1.0000x reference)
import jax
import jax.numpy as jnp
from jax import lax
from jax.experimental import pallas as pl
from jax.experimental.pallas import tpu as pltpu
from jax.experimental.pallas import tpu_sc as plsc

_NC, _NS, _L = 1, 16, 16
_NW = _NC * _NS


def _body(t_hbm, beta_hbm, alpha_hbm, out_b_hbm, out_a_hbm,
          idx_v, beta_v, alpha_v, ob_v, oa_v, sem_in, sem_out):
    wid = lax.axis_index("s") * _NC + lax.axis_index("c")
    bw = idx_v.shape[0]
    base = wid * bw
    n = beta_hbm.shape[0]
    cp_t = pltpu.async_copy(t_hbm.at[pl.ds(base, bw)], idx_v, sem_in)
    cp_b = pltpu.async_copy(beta_hbm, beta_v.at[pl.ds(0, n)], sem_in)
    cp_a = pltpu.async_copy(alpha_hbm, alpha_v.at[pl.ds(0, n)], sem_in)
    cp_t.wait()
    cp_b.wait()
    cp_a.wait()

    @plsc.parallel_loop(0, bw // _L, unroll=16)
    def _(i):
        off = i * _L
        idx = idx_v[pl.ds(off, _L)]
        ob_v[pl.ds(off, _L)] = plsc.load_gather(beta_v, [idx])
        oa_v[pl.ds(off, _L)] = plsc.load_gather(alpha_v, [idx])
    co_b = pltpu.async_copy(ob_v, out_b_hbm.at[pl.ds(base, bw)], sem_out)
    co_a = pltpu.async_copy(oa_v, out_a_hbm.at[pl.ds(base, bw)], sem_out)
    co_b.wait()
    co_a.wait()


def kernel(t, beta, alpha):
    b = t.shape[0]
    bw = b // _NW
    tbl_pad = (beta.shape[0] + _L - 1) // _L * _L
    run = pl.kernel(
        _body,
        out_type=(jax.ShapeDtypeStruct((b,), jnp.float32),
                  jax.ShapeDtypeStruct((b,), jnp.float32)),
        mesh=plsc.VectorSubcoreMesh(core_axis_name="c", subcore_axis_name="s",
                                    num_cores=_NC),
        scratch_types=[
            pltpu.VMEM((bw,), jnp.int32),
            pltpu.VMEM((tbl_pad,), jnp.float32),
            pltpu.VMEM((tbl_pad,), jnp.float32),
            pltpu.VMEM((bw,), jnp.float32),
            pltpu.VMEM((bw,), jnp.float32),
            pltpu.SemaphoreType.DMA,
            pltpu.SemaphoreType.DMA,
        ],
        compiler_params=pltpu.CompilerParams(needs_layout_passes=False),
    )
    return run(t, beta, alpha)


# speedup vs baseline: 1.0139x; 1.0139x over previous
import jax
import jax.numpy as jnp
from jax import lax
from jax.experimental import pallas as pl
from jax.experimental.pallas import tpu as pltpu
from jax.experimental.pallas import tpu_sc as plsc

_NC, _NS, _L = 1, 16, 16
_NW = _NC * _NS


def _body(t_hbm, beta_hbm, alpha_hbm, out_b_hbm, out_a_hbm,
          idx_v, beta_v, alpha_v, ob_v, oa_v, sem):
    wid = lax.axis_index("s") * _NC + lax.axis_index("c")
    bw = idx_v.shape[0]
    base = wid * bw
    n = beta_hbm.shape[0]
    cp_t = pltpu.async_copy(t_hbm.at[pl.ds(base, bw)], idx_v, sem)
    cp_b = pltpu.async_copy(beta_hbm, beta_v.at[pl.ds(0, n)], sem)
    cp_a = pltpu.async_copy(alpha_hbm, alpha_v.at[pl.ds(0, n)], sem)
    cp_t.wait()
    cp_b.wait()
    cp_a.wait()

    @plsc.parallel_loop(0, bw // _L, unroll=8)
    def _(i):
        off = i * _L
        idx = idx_v[pl.ds(off, _L)]
        ob_v[pl.ds(off, _L)] = plsc.load_gather(beta_v, [idx])
        oa_v[pl.ds(off, _L)] = plsc.load_gather(alpha_v, [idx])
    co_b = pltpu.async_copy(ob_v, out_b_hbm.at[pl.ds(base, bw)], sem)
    co_a = pltpu.async_copy(oa_v, out_a_hbm.at[pl.ds(base, bw)], sem)
    co_b.wait()
    co_a.wait()


def kernel(t, beta, alpha):
    b = t.shape[0]
    bw = b // _NW
    tbl_pad = (beta.shape[0] + _L - 1) // _L * _L
    run = pl.kernel(
        _body,
        out_type=(jax.ShapeDtypeStruct((b,), jnp.float32),
                  jax.ShapeDtypeStruct((b,), jnp.float32)),
        mesh=plsc.VectorSubcoreMesh(core_axis_name="c", subcore_axis_name="s",
                                    num_cores=_NC),
        scratch_types=[
            pltpu.VMEM((bw,), jnp.int32),
            pltpu.VMEM((tbl_pad,), jnp.float32),
            pltpu.VMEM((tbl_pad,), jnp.float32),
            pltpu.VMEM((bw,), jnp.float32),
            pltpu.VMEM((bw,), jnp.float32),
            pltpu.SemaphoreType.DMA,
        ],
        compiler_params=pltpu.CompilerParams(needs_layout_passes=False),
    )
    return run(t, beta, alpha)
